# Initial kernel scaffold; baseline (speedup 1.0000x reference)
import jax
import jax.numpy as jnp
from jax.experimental import pallas as pl


def kernel(query, Wq, bq, Wk, bk, Wv, bv, Wo, bo):
    raise NotImplementedError("write your pallas kernel here")


# speedup vs baseline: 1.1004x; 1.1004x over previous
import functools

import numpy as np
import jax
import jax.numpy as jnp
from jax.experimental import pallas as pl

N = 4096
F = N // 2 + 1
FP = 2176
B = 4
D = 1024
H = 16
DH = 64
KT = 64

_PREC = jax.lax.Precision.HIGHEST


@functools.lru_cache(maxsize=1)
def _consts():
    f = np.arange(FP, dtype=np.int64)[:, None]
    n = np.arange(N, dtype=np.int64)[None, :]
    ang = 2.0 * np.pi * ((f * n) % N).astype(np.float64) / N
    cos = np.cos(ang)
    sin = np.sin(ang)
    valid = (f < F).astype(np.float64)
    ccos = (cos * valid).astype(np.float32)
    csin = (sin * valid).astype(np.float32)
    wr = np.full((FP, 1), 2.0 / N)
    wr[0, 0] = 1.0 / N
    wr[N // 2, 0] = 1.0 / N
    wr = wr * valid
    brt = np.ascontiguousarray((cos * wr).T).astype(np.float32)
    bit = np.ascontiguousarray((-sin * wr).T).astype(np.float32)
    g = np.zeros((D * B, B * H), dtype=np.float32)
    for c in range(D * B):
        g[c, c // DH] = 1.0
    return ccos, csin, brt, bit, g


def _mm_bias_body(x_ref, w_ref, b_ref, o_ref, *, precision):
    acc = jax.lax.dot_general(
        x_ref[...], w_ref[...], (((1,), (0,)), ((), ())),
        preferred_element_type=jnp.float32, precision=precision)
    o_ref[...] = acc + b_ref[...]


def _mm_bias(x, w, b, bm, bn, precision=jax.lax.Precision.DEFAULT):
    m, k = x.shape
    _, nn = w.shape
    return pl.pallas_call(
        functools.partial(_mm_bias_body, precision=precision),
        grid=(nn // bn, m // bm),
        in_specs=[
            pl.BlockSpec((bm, k), lambda j, i: (i, 0)),
            pl.BlockSpec((k, bn), lambda j, i: (0, j)),
            pl.BlockSpec((1, bn), lambda j, i: (0, j)),
        ],
        out_specs=pl.BlockSpec((bm, bn), lambda j, i: (i, j)),
        out_shape=jax.ShapeDtypeStruct((m, nn), jnp.float32),
    )(x, w, b.reshape(1, nn))


def _dft_body(cc_ref, cs_ref, q_ref, k_ref, v_ref, qr_ref, qi_ref, kc_ref,
              vc_ref):
    cc = cc_ref[...]
    cs = cs_ref[...]
    dims = (((1,), (0,)), ((), ()))
    qr_ref[...] = jax.lax.dot_general(cc, q_ref[...], dims,
                                      preferred_element_type=jnp.float32,
                                      precision=_PREC)
    qi_ref[...] = -jax.lax.dot_general(cs, q_ref[...], dims,
                                       preferred_element_type=jnp.float32,
                                       precision=_PREC)
    kc_ref[...] = jax.lax.dot_general(cc, k_ref[...], dims,
                                      preferred_element_type=jnp.float32,
                                      precision=_PREC)
    vc_ref[...] = jax.lax.dot_general(cc, v_ref[...], dims,
                                      preferred_element_type=jnp.float32,
                                      precision=_PREC)


def _fwd_dft(ccos, csin, qt, kt, vt):
    bm, bn = 128, 256
    nn = qt.shape[1]
    spec_lhs = pl.BlockSpec((bm, N), lambda j, i: (i, 0))
    spec_rhs = pl.BlockSpec((N, bn), lambda j, i: (0, j))
    spec_out = pl.BlockSpec((bm, bn), lambda j, i: (i, j))
    outs = [jax.ShapeDtypeStruct((FP, nn), jnp.float32)] * 4
    return pl.pallas_call(
        _dft_body,
        grid=(nn // bn, FP // bm),
        in_specs=[spec_lhs, spec_lhs, spec_rhs, spec_rhs, spec_rhs],
        out_specs=[spec_out] * 4,
        out_shape=outs,
    )(ccos, csin, qt, kt, vt)


def _energy_body(qr_ref, qi_ref, g_ref, e_ref):
    qr = qr_ref[...]
    qi = qi_ref[...]
    mag = jnp.sqrt(qr * qr + qi * qi)
    e = jax.lax.dot_general(mag, g_ref[...], (((1,), (0,)), ((), ())),
                            preferred_element_type=jnp.float32,
                            precision=_PREC) * (1.0 / DH)
    row = jax.lax.broadcasted_iota(jnp.int32, e.shape, 0) + \
        pl.program_id(0) * qr.shape[0]
    e_ref[...] = jnp.where(row < F, e, -1e30)


def _energy(qr, qi, g):
    bm = 128
    nn = qr.shape[1]
    return pl.pallas_call(
        _energy_body,
        grid=(FP // bm,),
        in_specs=[
            pl.BlockSpec((bm, nn), lambda i: (i, 0)),
            pl.BlockSpec((bm, nn), lambda i: (i, 0)),
            pl.BlockSpec((nn, B * H), lambda i: (0, 0)),
        ],
        out_specs=pl.BlockSpec((bm, B * H), lambda i: (i, 0)),
        out_shape=jax.ShapeDtypeStruct((FP, B * H), jnp.float32),
    )(qr, qi, g)


def _topk_body(e_ref, idx_ref):
    iota_f = jax.lax.broadcasted_iota(jnp.int32, (B * H, FP), 1)
    iota_k = jax.lax.broadcasted_iota(jnp.int32, (B * H, KT), 1)

    def step(j, carry):
        e, out = carry
        m = jnp.max(e, axis=1, keepdims=True)
        cand = jnp.where(e >= m, iota_f, FP + 1)
        sel = jnp.min(cand, axis=1, keepdims=True)
        out = jnp.where(iota_k == j, sel, out)
        e = jnp.where(iota_f == sel, -jnp.inf, e)
        return e, out

    _, out = jax.lax.fori_loop(
        0, KT, step, (e_ref[...], jnp.zeros((B * H, KT), jnp.int32)))
    idx_ref[...] = out


def _topk(et):
    return pl.pallas_call(
        _topk_body,
        out_shape=jax.ShapeDtypeStruct((B * H, KT), jnp.int32),
    )(et)


def _attn_body(qr_ref, qi_ref, kc_ref, vc_ref, idx_ref, or_ref, oi_ref):
    fcol = jax.lax.broadcasted_iota(jnp.int32, (FP, KT), 0)
    frow = jax.lax.broadcasted_iota(jnp.int32, (FP, 1), 0)
    fmask = (frow < F).astype(jnp.float32)
    dims_t = (((0,), (0,)), ((), ()))
    dims_n = (((1,), (0,)), ((), ()))
    for hh in range(2):
        sl = slice(hh * DH, (hh + 1) * DH)
        qr = qr_ref[:, sl]
        qi = qi_ref[:, sl]
        kc = kc_ref[:, sl]
        vc = vc_ref[:, sl]
        idx = idx_ref[0, hh:hh + 1, :]
        ot = (fcol == idx).astype(jnp.float32)
        qt = jax.lax.dot_general(ot, qr, dims_t,
                                 preferred_element_type=jnp.float32,
                                 precision=_PREC)
        ktop = jax.lax.dot_general(ot, kc, dims_t,
                                   preferred_element_type=jnp.float32,
                                   precision=_PREC)
        vtop = jax.lax.dot_general(ot, vc, dims_t,
                                   preferred_element_type=jnp.float32,
                                   precision=_PREC)
        score = jnp.sum(qt * ktop, axis=1, keepdims=True) / (DH ** 0.5 + 1e-8)
        score = score - jnp.max(score, axis=0, keepdims=True)
        ex = jnp.exp(score)
        attn = ex / jnp.sum(ex, axis=0, keepdims=True)
        qmr = jnp.sum(qr * fmask, axis=0, keepdims=True) * (1.0 / F)
        qmi = jnp.sum(qi * fmask, axis=0, keepdims=True) * (1.0 / F)
        src = attn * vtop
        delta = src - qmr
        outr = qmr + jax.lax.dot_general(ot, delta, dims_n,
                                         preferred_element_type=jnp.float32,
                                         precision=_PREC)
        covered = jnp.sum(ot, axis=1, keepdims=True)
        outi = qmi * (1.0 - covered)
        or_ref[:, sl] = outr
        oi_ref[:, sl] = outi


def _attn(qr, qi, kc, vc, idx):
    nn = qr.shape[1]
    nblk = nn // 128
    spec_in = pl.BlockSpec((FP, 128), lambda i: (0, i))
    idx3 = idx.reshape(nblk, 2, KT)
    return pl.pallas_call(
        _attn_body,
        grid=(nblk,),
        in_specs=[spec_in, spec_in, spec_in, spec_in,
                  pl.BlockSpec((1, 2, KT), lambda i: (i, 0, 0))],
        out_specs=[spec_in, spec_in],
        out_shape=[jax.ShapeDtypeStruct((FP, nn), jnp.float32)] * 2,
    )(qr, qi, kc, vc, idx3)


def _mm2_body(a1_ref, x1_ref, a2_ref, x2_ref, o_ref):
    dims = (((1,), (0,)), ((), ()))
    acc = jax.lax.dot_general(a1_ref[...], x1_ref[...], dims,
                              preferred_element_type=jnp.float32,
                              precision=_PREC)
    acc += jax.lax.dot_general(a2_ref[...], x2_ref[...], dims,
                               preferred_element_type=jnp.float32,
                               precision=_PREC)
    o_ref[...] = acc


def _inv_dft(brt, bit, outr, outi):
    bm, bn = 512, 512
    nn = outr.shape[1]
    return pl.pallas_call(
        _mm2_body,
        grid=(N // bm, nn // bn),
        in_specs=[
            pl.BlockSpec((bm, FP), lambda i, j: (i, 0)),
            pl.BlockSpec((FP, bn), lambda i, j: (0, j)),
            pl.BlockSpec((bm, FP), lambda i, j: (i, 0)),
            pl.BlockSpec((FP, bn), lambda i, j: (0, j)),
        ],
        out_specs=pl.BlockSpec((bm, bn), lambda i, j: (i, j)),
        out_shape=jax.ShapeDtypeStruct((N, nn), jnp.float32),
    )(brt, outr, bit, outi)


def kernel(query, Wq, bq, Wk, bk, Wv, bv, Wo, bo):
    ccos, csin, brt, bit, g = _consts()
    ccos = jnp.asarray(ccos)
    csin = jnp.asarray(csin)
    brt = jnp.asarray(brt)
    bit = jnp.asarray(bit)
    g = jnp.asarray(g)

    x = query.reshape(B * N, D)
    w3 = jnp.concatenate([Wq.T, Wk.T, Wv.T], axis=1)
    b3 = jnp.concatenate([bq, bk, bv])
    qkv = _mm_bias(x, w3, b3, bm=512, bn=1024)

    def to_freq_layout(a):
        return a.reshape(B, N, D).transpose(1, 0, 2).reshape(N, B * D)

    qt = to_freq_layout(qkv[:, :D])
    kt = to_freq_layout(qkv[:, D:2 * D])
    vt = to_freq_layout(qkv[:, 2 * D:])

    qr, qi, kc, vc = _fwd_dft(ccos, csin, qt, kt, vt)
    e = _energy(qr, qi, g)
    idx = _topk(e.T)
    outr, outi = _attn(qr, qi, kc, vc, idx)
    t = _inv_dft(brt, bit, outr, outi)
    t = t.reshape(N, B, D).transpose(1, 0, 2).reshape(B * N, D)
    out = _mm_bias(t, Wo.T, bo, bm=512, bn=1024)
    return out.reshape(B, N, D)


# speedup vs baseline: 2.5576x; 2.3243x over previous
import functools

import numpy as np
import jax
import jax.numpy as jnp
from jax.experimental import pallas as pl

N = 4096
F = N // 2 + 1
FP = 2176
B = 4
D = 1024
H = 16
DH = 64
KT = 64

_PREC = jax.lax.Precision.DEFAULT


@functools.lru_cache(maxsize=1)
def _consts():
    f = np.arange(FP, dtype=np.int64)[:, None]
    n = np.arange(N, dtype=np.int64)[None, :]
    ang = 2.0 * np.pi * ((f * n) % N).astype(np.float64) / N
    cos = np.cos(ang)
    sin = np.sin(ang)
    valid = (f < F).astype(np.float64)
    ccos = (cos * valid).astype(np.float32)
    csin = (sin * valid).astype(np.float32)
    wr = np.full((FP, 1), 2.0 / N)
    wr[0, 0] = 1.0 / N
    wr[N // 2, 0] = 1.0 / N
    wr = wr * valid
    brt = np.ascontiguousarray((cos * wr).T).astype(np.float32)
    bit = np.ascontiguousarray((-sin * wr).T).astype(np.float32)
    g = np.zeros((D * B, B * H), dtype=np.float32)
    for c in range(D * B):
        g[c, c // DH] = 1.0
    return ccos, csin, brt, bit, g


def _mm_bias_body(x_ref, w_ref, b_ref, o_ref, *, precision):
    acc = jax.lax.dot_general(
        x_ref[...], w_ref[...], (((1,), (0,)), ((), ())),
        preferred_element_type=jnp.float32, precision=precision)
    o_ref[...] = acc + b_ref[...]


def _mm_bias(x, w, b, bm, bn, precision=jax.lax.Precision.DEFAULT):
    m, k = x.shape
    _, nn = w.shape
    return pl.pallas_call(
        functools.partial(_mm_bias_body, precision=precision),
        grid=(nn // bn, m // bm),
        in_specs=[
            pl.BlockSpec((bm, k), lambda j, i: (i, 0)),
            pl.BlockSpec((k, bn), lambda j, i: (0, j)),
            pl.BlockSpec((1, bn), lambda j, i: (0, j)),
        ],
        out_specs=pl.BlockSpec((bm, bn), lambda j, i: (i, j)),
        out_shape=jax.ShapeDtypeStruct((m, nn), jnp.float32),
    )(x, w, b.reshape(1, nn))


def _dft_body(cc_ref, cs_ref, q_ref, k_ref, v_ref, qr_ref, qi_ref, kc_ref,
              vc_ref):
    cc = cc_ref[...]
    cs = cs_ref[...]
    dims = (((1,), (0,)), ((), ()))
    qr_ref[...] = jax.lax.dot_general(cc, q_ref[...], dims,
                                      preferred_element_type=jnp.float32,
                                      precision=_PREC)
    qi_ref[...] = -jax.lax.dot_general(cs, q_ref[...], dims,
                                       preferred_element_type=jnp.float32,
                                       precision=_PREC)
    kc_ref[...] = jax.lax.dot_general(cc, k_ref[...], dims,
                                      preferred_element_type=jnp.float32,
                                      precision=_PREC)
    vc_ref[...] = jax.lax.dot_general(cc, v_ref[...], dims,
                                      preferred_element_type=jnp.float32,
                                      precision=_PREC)


def _fwd_dft(ccos, csin, qt, kt, vt):
    bm, bn = 128, 256
    nn = qt.shape[1]
    spec_lhs = pl.BlockSpec((bm, N), lambda j, i: (i, 0))
    spec_rhs = pl.BlockSpec((N, bn), lambda j, i: (0, j))
    spec_out = pl.BlockSpec((bm, bn), lambda j, i: (i, j))
    outs = [jax.ShapeDtypeStruct((FP, nn), jnp.float32)] * 4
    return pl.pallas_call(
        _dft_body,
        grid=(nn // bn, FP // bm),
        in_specs=[spec_lhs, spec_lhs, spec_rhs, spec_rhs, spec_rhs],
        out_specs=[spec_out] * 4,
        out_shape=outs,
    )(ccos, csin, qt, kt, vt)


def _energy_body(qr_ref, qi_ref, g_ref, e_ref):
    qr = qr_ref[...]
    qi = qi_ref[...]
    mag = jnp.sqrt(qr * qr + qi * qi)
    e = jax.lax.dot_general(mag, g_ref[...], (((1,), (0,)), ((), ())),
                            preferred_element_type=jnp.float32,
                            precision=_PREC) * (1.0 / DH)
    row = jax.lax.broadcasted_iota(jnp.int32, e.shape, 0) + \
        pl.program_id(0) * qr.shape[0]
    e_ref[...] = jnp.where(row < F, e, -1e30)


def _energy(qr, qi, g):
    bm = 128
    nn = qr.shape[1]
    return pl.pallas_call(
        _energy_body,
        grid=(FP // bm,),
        in_specs=[
            pl.BlockSpec((bm, nn), lambda i: (i, 0)),
            pl.BlockSpec((bm, nn), lambda i: (i, 0)),
            pl.BlockSpec((nn, B * H), lambda i: (0, 0)),
        ],
        out_specs=pl.BlockSpec((bm, B * H), lambda i: (i, 0)),
        out_shape=jax.ShapeDtypeStruct((FP, B * H), jnp.float32),
    )(qr, qi, g)


def _topk_body(e_ref, idx_ref):
    iota_f = jax.lax.broadcasted_iota(jnp.int32, (B * H, FP), 1)
    iota_k = jax.lax.broadcasted_iota(jnp.int32, (B * H, KT), 1)

    def step(j, carry):
        e, out = carry
        m = jnp.max(e, axis=1, keepdims=True)
        cand = jnp.where(e >= m, iota_f, FP + 1)
        sel = jnp.min(cand, axis=1, keepdims=True)
        out = jnp.where(iota_k == j, sel, out)
        e = jnp.where(iota_f == sel, -jnp.inf, e)
        return e, out

    _, out = jax.lax.fori_loop(
        0, KT, step, (e_ref[...], jnp.zeros((B * H, KT), jnp.int32)))
    idx_ref[...] = out


def _topk(et):
    return pl.pallas_call(
        _topk_body,
        out_shape=jax.ShapeDtypeStruct((B * H, KT), jnp.int32),
    )(et)


def _attn_body(qr_ref, qi_ref, kc_ref, vc_ref, idx_ref, or_ref, oi_ref):
    fcol = jax.lax.broadcasted_iota(jnp.int32, (FP, KT), 0)
    frow = jax.lax.broadcasted_iota(jnp.int32, (FP, 1), 0)
    fmask = (frow < F).astype(jnp.float32)
    dims_t = (((0,), (0,)), ((), ()))
    dims_n = (((1,), (0,)), ((), ()))
    for hh in range(2):
        sl = slice(hh * DH, (hh + 1) * DH)
        qr = qr_ref[:, sl]
        qi = qi_ref[:, sl]
        kc = kc_ref[:, sl]
        vc = vc_ref[:, sl]
        idx = idx_ref[0, hh:hh + 1, :]
        ot = (fcol == idx).astype(jnp.float32)
        qt = jax.lax.dot_general(ot, qr, dims_t,
                                 preferred_element_type=jnp.float32,
                                 precision=_PREC)
        ktop = jax.lax.dot_general(ot, kc, dims_t,
                                   preferred_element_type=jnp.float32,
                                   precision=_PREC)
        vtop = jax.lax.dot_general(ot, vc, dims_t,
                                   preferred_element_type=jnp.float32,
                                   precision=_PREC)
        score = jnp.sum(qt * ktop, axis=1, keepdims=True) / (DH ** 0.5 + 1e-8)
        score = score - jnp.max(score, axis=0, keepdims=True)
        ex = jnp.exp(score)
        attn = ex / jnp.sum(ex, axis=0, keepdims=True)
        qmr = jnp.sum(qr * fmask, axis=0, keepdims=True) * (1.0 / F)
        qmi = jnp.sum(qi * fmask, axis=0, keepdims=True) * (1.0 / F)
        src = attn * vtop
        delta = src - qmr
        outr = qmr + jax.lax.dot_general(ot, delta, dims_n,
                                         preferred_element_type=jnp.float32,
                                         precision=_PREC)
        covered = jnp.sum(ot, axis=1, keepdims=True)
        outi = qmi * (1.0 - covered)
        or_ref[:, sl] = outr
        oi_ref[:, sl] = outi


def _attn(qr, qi, kc, vc, idx):
    nn = qr.shape[1]
    nblk = nn // 128
    spec_in = pl.BlockSpec((FP, 128), lambda i: (0, i))
    idx3 = idx.reshape(nblk, 2, KT)
    return pl.pallas_call(
        _attn_body,
        grid=(nblk,),
        in_specs=[spec_in, spec_in, spec_in, spec_in,
                  pl.BlockSpec((1, 2, KT), lambda i: (i, 0, 0))],
        out_specs=[spec_in, spec_in],
        out_shape=[jax.ShapeDtypeStruct((FP, nn), jnp.float32)] * 2,
    )(qr, qi, kc, vc, idx3)


def _mm2_body(a1_ref, x1_ref, a2_ref, x2_ref, o_ref):
    dims = (((1,), (0,)), ((), ()))
    acc = jax.lax.dot_general(a1_ref[...], x1_ref[...], dims,
                              preferred_element_type=jnp.float32,
                              precision=_PREC)
    acc += jax.lax.dot_general(a2_ref[...], x2_ref[...], dims,
                               preferred_element_type=jnp.float32,
                               precision=_PREC)
    o_ref[...] = acc


def _inv_dft(brt, bit, outr, outi):
    bm, bn = 512, 512
    nn = outr.shape[1]
    return pl.pallas_call(
        _mm2_body,
        grid=(N // bm, nn // bn),
        in_specs=[
            pl.BlockSpec((bm, FP), lambda i, j: (i, 0)),
            pl.BlockSpec((FP, bn), lambda i, j: (0, j)),
            pl.BlockSpec((bm, FP), lambda i, j: (i, 0)),
            pl.BlockSpec((FP, bn), lambda i, j: (0, j)),
        ],
        out_specs=pl.BlockSpec((bm, bn), lambda i, j: (i, j)),
        out_shape=jax.ShapeDtypeStruct((N, nn), jnp.float32),
    )(brt, outr, bit, outi)


def kernel(query, Wq, bq, Wk, bk, Wv, bv, Wo, bo):
    ccos, csin, brt, bit, g = _consts()
    ccos = jnp.asarray(ccos)
    csin = jnp.asarray(csin)
    brt = jnp.asarray(brt)
    bit = jnp.asarray(bit)
    g = jnp.asarray(g)

    x = query.reshape(B * N, D)
    w3 = jnp.concatenate([Wq.T, Wk.T, Wv.T], axis=1)
    b3 = jnp.concatenate([bq, bk, bv])
    qkv = _mm_bias(x, w3, b3, bm=512, bn=1024)

    def to_freq_layout(a):
        return a.reshape(B, N, D).transpose(1, 0, 2).reshape(N, B * D)

    qt = to_freq_layout(qkv[:, :D])
    kt = to_freq_layout(qkv[:, D:2 * D])
    vt = to_freq_layout(qkv[:, 2 * D:])

    qr, qi, kc, vc = _fwd_dft(ccos, csin, qt, kt, vt)
    e = _energy(qr, qi, g)
    idx = _topk(e.T)
    outr, outi = _attn(qr, qi, kc, vc, idx)
    t = _inv_dft(brt, bit, outr, outi)
    t = t.reshape(N, B, D).transpose(1, 0, 2).reshape(B * N, D)
    out = _mm_bias(t, Wo.T, bo, bm=512, bn=1024)
    return out.reshape(B, N, D)
